# baseline (speedup 1.0000x reference)
import functools

import jax
import jax.numpy as jnp
from jax import lax
from jax.experimental import pallas as pl
from jax.experimental.pallas import tpu as pltpu
from jax.experimental.pallas import tpu_sc as plsc

MAX_LEN = 200
D_MODEL = 64
BATCH = 4096

NUM_CORES = 2
NUM_SUBCORES = 16
NUM_WORKERS = NUM_CORES * NUM_SUBCORES
ROWS_PER_WORKER = BATCH // NUM_WORKERS
REP = 8
BURSTS = ROWS_PER_WORKER // REP

_mesh = plsc.VectorSubcoreMesh(core_axis_name="c", subcore_axis_name="s")


@functools.partial(
    pl.kernel,
    mesh=_mesh,
    out_type=jax.ShapeDtypeStruct((BATCH, MAX_LEN, D_MODEL), jnp.float32),
    scratch_types=[pltpu.VMEM((REP, MAX_LEN, D_MODEL), jnp.float32)],
    compiler_params=pltpu.CompilerParams(use_tc_tiling_on_sc=False),
)
def _broadcast_table(table_hbm, out_hbm, buf):
    wid = lax.axis_index("s") * NUM_CORES + lax.axis_index("c")
    base = wid * ROWS_PER_WORKER
    for r in range(REP):
        pltpu.sync_copy(table_hbm, buf.at[r])

    def body(i, carry):
        pltpu.sync_copy(buf, out_hbm.at[pl.ds(base + i * REP, REP)])
        return carry

    lax.fori_loop(0, BURSTS, body, 0)


def kernel(x, pe_weight):
    del x
    return _broadcast_table(pe_weight)


# speedup vs baseline: 1.4366x; 1.4366x over previous
import functools

import jax
import jax.numpy as jnp
from jax import lax
from jax.experimental import pallas as pl
from jax.experimental.pallas import tpu as pltpu
from jax.experimental.pallas import tpu_sc as plsc

MAX_LEN = 200
D_MODEL = 64
BATCH = 4096

NUM_CORES = 2
NUM_SUBCORES = 16
NUM_WORKERS = NUM_CORES * NUM_SUBCORES
ROWS_PER_WORKER = BATCH // NUM_WORKERS
REP = 4
BURSTS = ROWS_PER_WORKER // REP

_mesh = plsc.VectorSubcoreMesh(core_axis_name="c", subcore_axis_name="s")


@functools.partial(
    pl.kernel,
    mesh=_mesh,
    out_type=jax.ShapeDtypeStruct((BATCH, MAX_LEN, D_MODEL), jnp.float32),
    scratch_types=[pltpu.VMEM((REP, MAX_LEN, D_MODEL), jnp.float32)],
)
def _broadcast_table(table_hbm, out_hbm, buf):
    wid = lax.axis_index("s") * NUM_CORES + lax.axis_index("c")
    base = wid * ROWS_PER_WORKER
    for r in range(REP):
        pltpu.sync_copy(table_hbm, buf.at[r])

    def body(i, carry):
        pltpu.sync_copy(buf, out_hbm.at[pl.ds(base + i * REP, REP)])
        return carry

    lax.fori_loop(0, BURSTS, body, 0)


B_BLK = 64


def _tc_body(table_ref, out_ref):
    out_ref[...] = jnp.broadcast_to(
        table_ref[...][None], (B_BLK, MAX_LEN, D_MODEL)
    )


_tc_call = pl.pallas_call(
    _tc_body,
    grid=(BATCH // B_BLK,),
    in_specs=[pl.BlockSpec((MAX_LEN, D_MODEL), lambda i: (0, 0))],
    out_specs=pl.BlockSpec((B_BLK, MAX_LEN, D_MODEL), lambda i: (i, 0, 0)),
    out_shape=jax.ShapeDtypeStruct((BATCH, MAX_LEN, D_MODEL), jnp.float32),
)


def kernel(x, pe_weight):
    del x
    return _tc_call(pe_weight)


# speedup vs baseline: 2.3647x; 1.6461x over previous
import functools

import jax
import jax.numpy as jnp
from jax import lax
from jax.experimental import pallas as pl
from jax.experimental.pallas import tpu as pltpu
from jax.experimental.pallas import tpu_sc as plsc

MAX_LEN = 200
D_MODEL = 64
BATCH = 4096

NUM_CORES = 2
NUM_SUBCORES = 16
NUM_WORKERS = NUM_CORES * NUM_SUBCORES
ROWS_PER_WORKER = BATCH // NUM_WORKERS
REP = 4
BURSTS = ROWS_PER_WORKER // REP

_mesh = plsc.VectorSubcoreMesh(core_axis_name="c", subcore_axis_name="s")


@functools.partial(
    pl.kernel,
    mesh=_mesh,
    out_type=jax.ShapeDtypeStruct((BATCH, MAX_LEN, D_MODEL), jnp.float32),
    scratch_types=[pltpu.VMEM((REP, MAX_LEN, D_MODEL), jnp.float32)],
)
def _broadcast_table(table_hbm, out_hbm, buf):
    wid = lax.axis_index("s") * NUM_CORES + lax.axis_index("c")
    base = wid * ROWS_PER_WORKER
    for r in range(REP):
        pltpu.sync_copy(table_hbm, buf.at[r])

    def body(i, carry):
        pltpu.sync_copy(buf, out_hbm.at[pl.ds(base + i * REP, REP)])
        return carry

    lax.fori_loop(0, BURSTS, body, 0)


B_BLK = 64
ROW = MAX_LEN * D_MODEL


def _tc_body(table_ref, out_ref):
    out_ref[...] = jnp.broadcast_to(table_ref[...], (B_BLK, ROW))


_tc_call = pl.pallas_call(
    _tc_body,
    grid=(BATCH // B_BLK,),
    in_specs=[pl.BlockSpec((1, ROW), lambda i: (0, 0))],
    out_specs=pl.BlockSpec((B_BLK, ROW), lambda i: (i, 0)),
    out_shape=jax.ShapeDtypeStruct((BATCH, ROW), jnp.float32),
)


def kernel(x, pe_weight):
    del x
    flat = _tc_call(pe_weight.reshape(1, ROW))
    return flat.reshape(BATCH, MAX_LEN, D_MODEL)
